# baseline (speedup 1.0000x reference)
import functools

import jax
import jax.numpy as jnp
from jax import lax
from jax.experimental import pallas as pl
from jax.experimental.pallas import tpu as pltpu
from jax.experimental.pallas import tpu_sc as plsc

_S0 = 32
_V0 = 16
_AVG_NUM_NEIGHBORS = 16.0

_CHUNK = 128
_DG = 80


def _sc_gather(nf_tab, s_table, senders, receivers):
    E = senders.shape[0]
    info = plsc.get_sparse_core_info()
    nw = info.num_cores * info.num_subcores
    n_chunks = E // _CHUNK
    per_w = n_chunks // nw
    n_pairs = per_w // 2
    mesh = plsc.VectorSubcoreMesh(core_axis_name="c", subcore_axis_name="s")

    @functools.partial(
        pl.kernel,
        out_type=[
            jax.ShapeDtypeStruct((E, _DG), jnp.float32),
            jax.ShapeDtypeStruct((E, _S0), jnp.float32),
        ],
        mesh=mesh,
        scratch_types=[
            pltpu.VMEM((_CHUNK,), jnp.int32),
            pltpu.VMEM((_CHUNK,), jnp.int32),
            pltpu.VMEM((_CHUNK, _DG), jnp.float32),
            pltpu.VMEM((_CHUNK, _S0), jnp.float32),
            pltpu.VMEM((_CHUNK,), jnp.int32),
            pltpu.VMEM((_CHUNK,), jnp.int32),
            pltpu.VMEM((_CHUNK, _DG), jnp.float32),
            pltpu.VMEM((_CHUNK, _S0), jnp.float32),
            pltpu.SemaphoreType.DMA,
            pltpu.SemaphoreType.DMA,
            pltpu.SemaphoreType.DMA,
            pltpu.SemaphoreType.DMA,
        ],
        compiler_params=pltpu.CompilerParams(use_tc_tiling_on_sc=False),
    )
    def gather_kernel(nf_hbm, st_hbm, snd_hbm, rcv_hbm, send_out, recv_out,
                      sidx0, ridx0, send0, recv0,
                      sidx1, ridx1, send1, recv1,
                      semA0, semB0, semA1, semB1):
        wid = lax.axis_index("s") * info.num_cores + lax.axis_index("c")

        slots = ((sidx0, ridx0, send0, recv0, semA0, semB0),
                 (sidx1, ridx1, send1, recv1, semA1, semB1))

        def start(i, slot):
            sidx, ridx, send_v, recv_v, semA, semB = slot
            off = (wid + i * nw) * _CHUNK
            pltpu.sync_copy(snd_hbm.at[pl.ds(off, _CHUNK)], sidx)
            pltpu.sync_copy(rcv_hbm.at[pl.ds(off, _CHUNK)], ridx)
            pltpu.async_copy(nf_hbm.at[sidx, :], send_v, semA)
            pltpu.async_copy(st_hbm.at[ridx, :], recv_v, semB)

        def finish(i, slot):
            sidx, ridx, send_v, recv_v, semA, semB = slot
            off = (wid + i * nw) * _CHUNK
            pltpu.make_async_copy(nf_hbm.at[sidx, :], send_v, semA).wait()
            pltpu.make_async_copy(st_hbm.at[ridx, :], recv_v, semB).wait()
            pltpu.sync_copy(send_v, send_out.at[pl.ds(off, _CHUNK)])
            pltpu.sync_copy(recv_v, recv_out.at[pl.ds(off, _CHUNK)])

        start(0, slots[0])

        def body(j, carry):
            i0 = 2 * j
            start(i0 + 1, slots[1])
            finish(i0, slots[0])
            start(i0 + 2, slots[0])
            finish(i0 + 1, slots[1])
            return carry

        lax.fori_loop(0, n_pairs - 1, body, 0)
        i0 = 2 * (n_pairs - 1)
        start(i0 + 1, slots[1])
        finish(i0, slots[0])
        finish(i0 + 1, slots[1])

    return gather_kernel(nf_tab, s_table, senders, receivers)


def _dense_body(g_ref, sr_ref, ev_ref, sef_ref, len_ref,
                w1a_ref, w1b_ref, w1c_ref, w1d_ref, w2_ref, out_ref):
    g = g_ref[...]
    ms = g[:, :_S0]
    mv = g[:, _S0:_S0 + 3 * _V0]
    sr = sr_ref[...]
    ev = ev_ref[...]
    sef = sef_ref[...]
    ln = len_ref[...]

    pre = (ms @ w1a_ref[...] + sr @ w1b_ref[...]
           + sef @ w1c_ref[...] + ln @ w1d_ref[...])
    pre = pre * (1.0 / jnp.sqrt(73.0))
    h = pre * jax.nn.sigmoid(pre)
    mix = (h @ w2_ref[...]) * (1.0 / 8.0)
    mix0 = mix[:, :_V0]
    mix1 = mix[:, _V0:]

    r3 = lax.broadcasted_iota(jnp.int32, (3, 3 * _V0), 0)
    c3 = lax.broadcasted_iota(jnp.int32, (3, 3 * _V0), 1)
    t48 = (c3 % 3 == r3).astype(jnp.float32)
    r9 = lax.broadcasted_iota(jnp.int32, (3, 3 * _S0), 0)
    c9 = lax.broadcasted_iota(jnp.int32, (3, 3 * _S0), 1)
    t96 = (c9 % 3 == r9).astype(jnp.float32)
    rr = lax.broadcasted_iota(jnp.int32, (3 * _V0, _V0), 0)
    rc = lax.broadcasted_iota(jnp.int32, (3 * _V0, _V0), 1)
    rmat = (rr // 3 == rc).astype(jnp.float32)
    ur = lax.broadcasted_iota(jnp.int32, (_S0, 3 * _S0), 0)
    uc = lax.broadcasted_iota(jnp.int32, (_S0, 3 * _S0), 1)
    umat = (uc // 3 == ur).astype(jnp.float32)

    ev48 = ev @ t48
    ev96 = ev @ t96
    out0 = ((mv * ev48) @ rmat) * (1.0 / jnp.sqrt(3.0)) * mix0
    out1 = ((ms * mix1) @ umat) * ev96
    out_ref[...] = jnp.concatenate([out0, out1], axis=1)


def _edge_messages(g, sr, ev, sef, lengths, W1, W2):
    E = g.shape[0]
    BE = 4096
    grid = (E // BE,)
    w1a = W1[:_S0]
    w1b = W1[_S0:2 * _S0]
    w1c = W1[2 * _S0:2 * _S0 + 8]
    w1d = W1[2 * _S0 + 8:]
    edge_spec = lambda w: pl.BlockSpec((BE, w), lambda i: (i, 0))
    full = lambda a: pl.BlockSpec(a.shape, lambda i: (0, 0))
    return pl.pallas_call(
        _dense_body,
        grid=grid,
        in_specs=[edge_spec(_DG), edge_spec(_S0),
                  edge_spec(3), edge_spec(8), edge_spec(1),
                  full(w1a), full(w1b), full(w1c), full(w1d), full(W2)],
        out_specs=edge_spec(_V0 + 3 * _S0),
        out_shape=jax.ShapeDtypeStruct((E, _V0 + 3 * _S0), jnp.float32),
    )(g, sr, ev, sef, lengths, w1a, w1b, w1c, w1d, W2)


def kernel(node_feats, edge_feats, scalar_edge_feats, lengths, senders, receivers, W1, W2):
    N = node_feats.shape[0]
    E = senders.shape[0]
    s_table = jnp.concatenate(
        [node_feats[:, :_S0], jnp.zeros((8, _S0), jnp.float32)], axis=0)
    K = 7
    Ec = -(-E // (K * 2 * 32 * _CHUNK)) * (2 * 32 * _CHUNK)
    pad = K * Ec - E
    snd_p = jnp.concatenate([senders, jnp.zeros((pad,), jnp.int32)])
    rcv_p = jnp.concatenate([receivers, jnp.full((pad,), N, jnp.int32)])
    ev_p = jnp.concatenate(
        [edge_feats[:, 1:4], jnp.zeros((pad, 3), jnp.float32)])
    sef_p = jnp.concatenate(
        [scalar_edge_feats, jnp.zeros((pad, scalar_edge_feats.shape[1]),
                                      jnp.float32)])
    ln_p = jnp.concatenate([lengths, jnp.zeros((pad, 1), jnp.float32)])
    agg = jnp.zeros((N + 8, _V0 + 3 * _S0), jnp.float32)
    for k in range(K):
        sl = slice(k * Ec, (k + 1) * Ec)
        g, sr = _sc_gather(node_feats, s_table, snd_p[sl], rcv_p[sl])
        msg = _edge_messages(g, sr, ev_p[sl], sef_p[sl], ln_p[sl], W1, W2)
        agg = agg.at[rcv_p[sl]].add(msg)
    return agg[:N] / _AVG_NUM_NEIGHBORS


# speedup vs baseline: 1.6023x; 1.6023x over previous
import functools

import jax
import jax.numpy as jnp
from jax import lax
from jax.experimental import pallas as pl
from jax.experimental.pallas import tpu as pltpu
from jax.experimental.pallas import tpu_sc as plsc

_S0 = 32
_V0 = 16
_AVG_NUM_NEIGHBORS = 16.0

_CHUNK = 128
_DG = 80


def _sc_gather(nf96, s_table, senders, receivers):
    E = senders.shape[0]
    info = plsc.get_sparse_core_info()
    nw = info.num_cores * info.num_subcores
    n_chunks = E // _CHUNK
    mesh = plsc.VectorSubcoreMesh(core_axis_name="c", subcore_axis_name="s")

    @functools.partial(
        pl.kernel,
        out_type=[
            jax.ShapeDtypeStruct((E, _DG), jnp.float32),
            jax.ShapeDtypeStruct((E, _S0), jnp.float32),
        ],
        mesh=mesh,
        scratch_types=[
            pltpu.VMEM((_CHUNK,), jnp.int32),
            pltpu.VMEM((_CHUNK,), jnp.int32),
            pltpu.VMEM((_CHUNK, _DG), jnp.float32),
            pltpu.VMEM((_CHUNK, _S0), jnp.float32),
            pltpu.SemaphoreType.DMA,
            pltpu.SemaphoreType.DMA,
        ],
        compiler_params=pltpu.CompilerParams(use_tc_tiling_on_sc=False),
    )
    def gather_kernel(nf_hbm, st_hbm, snd_hbm, rcv_hbm, send_out, recv_out,
                      sidx_v, ridx_v, send_v, recv_v, sem1, sem2):
        wid = lax.axis_index("s") * info.num_cores + lax.axis_index("c")
        n_mine = (n_chunks - wid + nw - 1) // nw

        def body(i, carry):
            g = wid + i * nw
            off = g * _CHUNK
            pltpu.sync_copy(snd_hbm.at[pl.ds(off, _CHUNK)], sidx_v)
            pltpu.sync_copy(rcv_hbm.at[pl.ds(off, _CHUNK)], ridx_v)
            c1 = pltpu.async_copy(nf_hbm.at[sidx_v, :], send_v, sem1)
            c2 = pltpu.async_copy(st_hbm.at[ridx_v, :], recv_v, sem2)
            c1.wait()
            c2.wait()
            pltpu.sync_copy(send_v, send_out.at[pl.ds(off, _CHUNK)])
            pltpu.sync_copy(recv_v, recv_out.at[pl.ds(off, _CHUNK)])
            return carry

        lax.fori_loop(0, n_mine, body, 0)

    return gather_kernel(nf96, s_table, senders, receivers)


def _dense_body(g_ref, sr_ref, ev_ref, sef_ref, len_ref,
                w1a_ref, w1b_ref, w1c_ref, w1d_ref, w2_ref, out_ref):
    g = g_ref[...]
    ms = g[:, :_S0]
    mv = g[:, _S0:_S0 + 3 * _V0]
    sr = sr_ref[...]
    ev = ev_ref[...]
    sef = sef_ref[...]
    ln = len_ref[...]

    pre = (ms @ w1a_ref[...] + sr @ w1b_ref[...]
           + sef @ w1c_ref[...] + ln @ w1d_ref[...])
    pre = pre * (1.0 / jnp.sqrt(73.0))
    h = pre * jax.nn.sigmoid(pre)
    mix = (h @ w2_ref[...]) * (1.0 / 8.0)
    mix0 = mix[:, :_V0]
    mix1 = mix[:, _V0:]

    r3 = lax.broadcasted_iota(jnp.int32, (3, 3 * _V0), 0)
    c3 = lax.broadcasted_iota(jnp.int32, (3, 3 * _V0), 1)
    t48 = (c3 % 3 == r3).astype(jnp.float32)
    r9 = lax.broadcasted_iota(jnp.int32, (3, 3 * _S0), 0)
    c9 = lax.broadcasted_iota(jnp.int32, (3, 3 * _S0), 1)
    t96 = (c9 % 3 == r9).astype(jnp.float32)
    rr = lax.broadcasted_iota(jnp.int32, (3 * _V0, _V0), 0)
    rc = lax.broadcasted_iota(jnp.int32, (3 * _V0, _V0), 1)
    rmat = (rr // 3 == rc).astype(jnp.float32)
    ur = lax.broadcasted_iota(jnp.int32, (_S0, 3 * _S0), 0)
    uc = lax.broadcasted_iota(jnp.int32, (_S0, 3 * _S0), 1)
    umat = (uc // 3 == ur).astype(jnp.float32)

    ev48 = ev @ t48
    ev96 = ev @ t96
    out0 = ((mv * ev48) @ rmat) * (1.0 / jnp.sqrt(3.0)) * mix0
    out1 = ((ms * mix1) @ umat) * ev96
    out_ref[...] = jnp.concatenate([out0, out1], axis=1)


def _edge_messages(g, sr, ev, sef, lengths, W1, W2):
    E = g.shape[0]
    BE = 4000
    grid = (E // BE,)
    w1a = W1[:_S0]
    w1b = W1[_S0:2 * _S0]
    w1c = W1[2 * _S0:2 * _S0 + 8]
    w1d = W1[2 * _S0 + 8:]
    edge_spec = lambda w: pl.BlockSpec((BE, w), lambda i: (i, 0))
    full = lambda a: pl.BlockSpec(a.shape, lambda i: (0, 0))
    return pl.pallas_call(
        _dense_body,
        grid=grid,
        in_specs=[edge_spec(_DG), edge_spec(_S0),
                  edge_spec(3), edge_spec(8), edge_spec(1),
                  full(w1a), full(w1b), full(w1c), full(w1d), full(W2)],
        out_specs=edge_spec(_V0 + 3 * _S0),
        out_shape=jax.ShapeDtypeStruct((E, _V0 + 3 * _S0), jnp.float32),
    )(g, sr, ev, sef, lengths, w1a, w1b, w1c, w1d, W2)


def kernel(node_feats, edge_feats, scalar_edge_feats, lengths, senders, receivers, W1, W2):
    N = node_feats.shape[0]
    E = senders.shape[0]
    nf96 = node_feats
    s_table = jnp.concatenate(
        [node_feats[:, :_S0], jnp.zeros((8, _S0), jnp.float32)], axis=0)
    ev = edge_feats[:, 1:4]
    K = 5
    Ec = E // K
    agg = jnp.zeros((N, _V0 + 3 * _S0), jnp.float32)
    for k in range(K):
        sl = slice(k * Ec, (k + 1) * Ec)
        g, sr = _sc_gather(nf96, s_table, senders[sl], receivers[sl])
        msg = _edge_messages(g, sr, ev[sl], scalar_edge_feats[sl],
                             lengths[sl], W1, W2)
        agg = agg.at[receivers[sl]].add(msg)
    return agg / _AVG_NUM_NEIGHBORS
